# baseline (speedup 1.0000x reference)
import functools

import jax
import jax.numpy as jnp
from jax import lax
from jax.experimental import pallas as pl
from jax.experimental.pallas import tpu as pltpu
from jax.experimental.pallas import tpu_sc as plsc

EMB_DIM = 32
BATCH_N = 16384
_NC, _NS = 2, 16
_NW = _NC * _NS
_BPW = BATCH_N // _NW
_BB = 2048


def _gather_body(u_hbm, i_hbm, ug_t, ig_t, um_t, im_t,
                 ug_o, ig_o, um_o, im_o,
                 idx_u, idx_i, r_ug, r_ig, r_um, r_im, sem):
    wid = lax.axis_index("s") * _NC + lax.axis_index("c")
    base = wid * _BPW
    pltpu.sync_copy(u_hbm.at[pl.ds(base, _BPW)], idx_u)
    pltpu.sync_copy(i_hbm.at[pl.ds(base, _BPW)], idx_i)
    cp1 = pltpu.async_copy(ug_t.at[idx_u], r_ug, sem)
    cp2 = pltpu.async_copy(ig_t.at[idx_i], r_ig, sem)
    cp3 = pltpu.async_copy(um_t.at[idx_u], r_um, sem)
    cp4 = pltpu.async_copy(im_t.at[idx_i], r_im, sem)
    cp1.wait()
    pltpu.sync_copy(r_ug, ug_o.at[pl.ds(base, _BPW)])
    cp2.wait()
    pltpu.sync_copy(r_ig, ig_o.at[pl.ds(base, _BPW)])
    cp3.wait()
    pltpu.sync_copy(r_um, um_o.at[pl.ds(base, _BPW)])
    cp4.wait()
    pltpu.sync_copy(r_im, im_o.at[pl.ds(base, _BPW)])


def _sc_gather(u, i, ug_t, ig_t, um_t, im_t):
    row = jax.ShapeDtypeStruct((BATCH_N, EMB_DIM), jnp.float32)
    mesh = plsc.VectorSubcoreMesh(core_axis_name="c", subcore_axis_name="s")
    return pl.kernel(
        _gather_body,
        out_type=(row, row, row, row),
        mesh=mesh,
        compiler_params=pltpu.CompilerParams(use_tc_tiling_on_sc=False),
        scratch_types=[
            pltpu.VMEM((_BPW,), jnp.int32),
            pltpu.VMEM((_BPW,), jnp.int32),
            pltpu.VMEM((_BPW, EMB_DIM), jnp.float32),
            pltpu.VMEM((_BPW, EMB_DIM), jnp.float32),
            pltpu.VMEM((_BPW, EMB_DIM), jnp.float32),
            pltpu.VMEM((_BPW, EMB_DIM), jnp.float32),
            pltpu.SemaphoreType.DMA,
        ],
    )(u, i, ug_t, ig_t, um_t, im_t)


def _mlp_body(ug, ig, um, im, W0, b0, W1, b1, W2, b2, W3, b3, Wp, bp, out):
    h = jnp.concatenate([um[...], im[...]], axis=-1)
    for W, b in ((W0, b0), (W1, b1), (W2, b2), (W3, b3)):
        h = jnp.maximum(
            lax.dot_general(h, W[...], (((1,), (1,)), ((), ())),
                            preferred_element_type=jnp.float32) + b[...],
            0.0)
    g = ug[...] * ig[...]
    c = jnp.concatenate([g, h], axis=-1)
    logits = jnp.sum(c * Wp[...], axis=1, keepdims=True) + bp[...].reshape(1, 1)
    out[...] = jax.nn.sigmoid(logits)


def _tc_mlp(ug, ig, um, im, W0, b0, W1, b1, W2, b2, W3, b3, Wp, bp,
            interpret=False):
    act = pl.BlockSpec((_BB, EMB_DIM), lambda b: (b, 0))
    full2 = lambda a: pl.BlockSpec(a.shape, lambda b: (0,) * a.ndim)
    grid = BATCH_N // _BB
    return pl.pallas_call(
        _mlp_body,
        grid=(grid,),
        in_specs=[act, act, act, act,
                  full2(W0), full2(b0), full2(W1), full2(b1),
                  full2(W2), full2(b2), full2(W3), full2(b3),
                  full2(Wp), full2(bp)],
        out_specs=pl.BlockSpec((_BB, 1), lambda b: (b, 0)),
        out_shape=jax.ShapeDtypeStruct((BATCH_N, 1), jnp.float32),
        interpret=interpret,
    )(ug, ig, um, im, W0, b0, W1, b1, W2, b2, W3, b3, Wp, bp)


def kernel(u, i, user_gmf, item_gmf, user_mlp, item_mlp,
           W0, b0, W1, b1, W2, b2, W3, b3, Wp, bp):
    u = u.astype(jnp.int32)
    i = i.astype(jnp.int32)
    ug, ig, um, im = _sc_gather(u, i, user_gmf, item_gmf, user_mlp, item_mlp)
    out = _tc_mlp(ug, ig, um, im, W0, b0, W1, b1, W2, b2, W3, b3, Wp, bp)
    return jnp.squeeze(out, axis=-1)
